# baseline (speedup 1.0000x reference)
import functools

import jax
import jax.numpy as jnp
from jax import lax
from jax.experimental import pallas as pl
from jax.experimental.pallas import tpu as pltpu
from jax.experimental.pallas import tpu_sc as plsc

N = 100000
NN = 50000
K = 3
M = 18
E = NN * M

NC, NS, L = 2, 16, 16
NW = NC * NS

CH = 128
CA = 128
NP = 106496
VT = NP // NW
NNP = 53248
NT = NNP // NW
EP = 901120
ET = EP // NW
SRROWS = 1600

_mesh = plsc.VectorSubcoreMesh(
    core_axis_name="c", subcore_axis_name="s", num_cores=NC, num_subcores=NS)
_sc_params = pltpu.CompilerParams(
    use_tc_tiling_on_sc=False, needs_layout_passes=False)


def _wid():
    return lax.axis_index("s") * NC + lax.axis_index("c")


@functools.partial(
    pl.kernel,
    out_type=jax.ShapeDtypeStruct((NNP, 4), jnp.float32),
    mesh=_mesh,
    compiler_params=_sc_params,
    scratch_types=[
        pltpu.VMEM((NT,), jnp.int32),
        pltpu.VMEM((CH, 4), jnp.float32),
        pltpu.VMEM((CH, 4), jnp.float32),
        pltpu.SemaphoreType.DMA,
        pltpu.SemaphoreType.DMA,
    ],
)
def _gather_node_rows(vp_hbm, nidx_hbm, out_hbm, idx_v, rva, rvb, sema, semb):
    base = _wid() * NT
    rows = [rva, rvb]
    sems = [sema, semb]
    pltpu.sync_copy(nidx_hbm.at[pl.ds(base, NT)], idx_v)
    nch = NT // CH

    def issue(i, s):
        pltpu.async_copy(vp_hbm.at[idx_v.at[pl.ds(i * CH, CH)]], rows[s],
                         sems[s])

    def drain(s):
        pltpu.make_async_copy(vp_hbm.at[idx_v.at[pl.ds(0, CH)]], rows[s],
                              sems[s]).wait()

    issue(0, 0)

    def body(j, carry):
        i = 2 * j
        issue(i + 1, 1)
        drain(0)
        pltpu.sync_copy(rows[0], out_hbm.at[pl.ds(base + i * CH, CH)])
        inext = jnp.minimum(i + 2, nch - 1)
        issue(inext, 0)
        drain(1)
        pltpu.sync_copy(rows[1], out_hbm.at[pl.ds(base + (i + 1) * CH, CH)])
        return carry

    lax.fori_loop(0, nch // 2, body, 0)
    drain(0)
    pltpu.sync_copy(rows[0], out_hbm.at[pl.ds(base + (nch - 1) * CH, CH)])


def _table_body(r_ref, t_ref, n_ref, o_ref):
    rx, ry, rz = r_ref[0:1, :], r_ref[1:2, :], r_ref[2:3, :]
    tx, ty, tz = t_ref[0:1, :], t_ref[1:2, :], t_ref[2:3, :]
    nx, ny, nz = n_ref[0:1, :], n_ref[1:2, :], n_ref[2:3, :]
    th = jnp.sqrt(rx * rx + ry * ry + rz * rz) + 1e-8
    inv = 1.0 / th
    x, y, z = rx * inv, ry * inv, rz * inv
    s = jnp.sin(th)
    cc = 1.0 - jnp.cos(th)
    r00 = 1.0 - cc * (y * y + z * z)
    r01 = -s * z + cc * x * y
    r02 = s * y + cc * x * z
    r10 = s * z + cc * x * y
    r11 = 1.0 - cc * (x * x + z * z)
    r12 = -s * x + cc * y * z
    r20 = -s * y + cc * x * z
    r21 = s * x + cc * y * z
    r22 = 1.0 - cc * (x * x + y * y)
    cx = nx + tx - (r00 * nx + r01 * ny + r02 * nz)
    cy = ny + ty - (r10 * nx + r11 * ny + r12 * nz)
    cz = nz + tz - (r20 * nx + r21 * ny + r22 * nz)
    cols = [cx, cy, cz, r00, r01, r02, r10, r11, r12, r20, r21, r22,
            nx, ny, nz, jnp.zeros_like(cx)]
    for i, col in enumerate(cols):
        o_ref[i:i + 1, :] = col


_build_table = pl.pallas_call(
    _table_body,
    out_shape=jax.ShapeDtypeStruct((16, NNP), jnp.float32),
)


def _lanes(g):
    return lax.iota(jnp.int32, L) + (g * L)


def _col(c):
    return jnp.full((L,), c, jnp.int32)


def _main_body(t_hbm, iw_hbm, whv_hbm, orn_hbm,
               ow_hbm, lp_hbm,
               ibw, nb,
               wtaba, wtabb,
               tabna, tabnb, srt,
               wb0, wb1, wb2, vbx, vby, vbz, obx, oby, obz,
               acc_a, acc_s, semwa, semwb, semna, semnb):
    wid = _wid()
    wtab = [wtaba, wtabb]
    wsem = [semwa, semwb]
    ntab = [tabna, tabnb]
    nsem = [semna, semnb]

    vbase = wid * VT
    pltpu.sync_copy(iw_hbm.at[pl.ds(vbase * 3, VT * 3)], ibw)
    pltpu.sync_copy(whv_hbm.at[0, pl.ds(vbase, VT)], wb0)
    pltpu.sync_copy(whv_hbm.at[1, pl.ds(vbase, VT)], wb1)
    pltpu.sync_copy(whv_hbm.at[2, pl.ds(vbase, VT)], wb2)
    pltpu.sync_copy(whv_hbm.at[3, pl.ds(vbase, VT)], vbx)
    pltpu.sync_copy(whv_hbm.at[4, pl.ds(vbase, VT)], vby)
    pltpu.sync_copy(whv_hbm.at[5, pl.ds(vbase, VT)], vbz)

    def warp_issue(i, s):
        pltpu.async_copy(t_hbm.at[ibw.at[pl.ds(i * CH * 3, CH * 3)]],
                         wtab[s], wsem[s])

    def warp_wait(s):
        pltpu.make_async_copy(t_hbm.at[ibw.at[pl.ds(0, CH * 3)]],
                              wtab[s], wsem[s]).wait()

    def warp_compute(i, s):
        tab = wtab[s]
        wbs = [wb0, wb1, wb2]
        for g in range(CH // L):
            ln = _lanes(g)
            off = i * CH + g * L
            a = [None] * 9
            bx = by = bz = None
            for k in range(3):
                lnk = ln * 3 + k
                w = wbs[k][pl.ds(off, L)]
                cx = plsc.load_gather(tab, [lnk, _col(0)])
                cy = plsc.load_gather(tab, [lnk, _col(1)])
                cz = plsc.load_gather(tab, [lnk, _col(2)])
                if k == 0:
                    bx, by, bz = w * cx, w * cy, w * cz
                else:
                    bx, by, bz = bx + w * cx, by + w * cy, bz + w * cz
                for j in range(9):
                    r = plsc.load_gather(tab, [lnk, _col(3 + j)])
                    a[j] = w * r if k == 0 else a[j] + w * r
            vx = vbx[pl.ds(off, L)]
            vy = vby[pl.ds(off, L)]
            vz = vbz[pl.ds(off, L)]
            obx[pl.ds(off, L)] = a[0] * vx + a[1] * vy + a[2] * vz + bx
            oby[pl.ds(off, L)] = a[3] * vx + a[4] * vy + a[5] * vz + by
            obz[pl.ds(off, L)] = a[6] * vx + a[7] * vy + a[8] * vz + bz

    nwc = VT // CH
    warp_issue(0, 0)

    def warp_pair(j, carry):
        i = 2 * j
        warp_issue(i + 1, 1)
        warp_wait(0)
        warp_compute(i, 0)
        inext = jnp.minimum(i + 2, nwc - 1)
        warp_issue(inext, 0)
        warp_wait(1)
        warp_compute(i + 1, 1)
        return carry

    lax.fori_loop(0, nwc // 2, warp_pair, 0)
    warp_wait(0)
    pltpu.sync_copy(obx, ow_hbm.at[0, pl.ds(vbase, VT)])
    pltpu.sync_copy(oby, ow_hbm.at[1, pl.ds(vbase, VT)])
    pltpu.sync_copy(obz, ow_hbm.at[2, pl.ds(vbase, VT)])

    acc_a[...] = jnp.zeros((L,), jnp.float32)
    acc_s[...] = jnp.zeros((L,), jnp.float32)
    ebase = wid * ET
    r0 = ebase // M
    pltpu.sync_copy(t_hbm.at[pl.ds(r0, SRROWS)], srt)
    pltpu.sync_copy(orn_hbm.at[pl.ds(ebase, ET)], nb)

    def arap_issue(i, s):
        pltpu.async_copy(t_hbm.at[nb.at[pl.ds(i * CA, CA)]], ntab[s],
                         nsem[s])

    def arap_wait(s):
        pltpu.make_async_copy(t_hbm.at[nb.at[pl.ds(0, CA)]], ntab[s],
                              nsem[s]).wait()

    def arap_compute(i, s):
        tabn = ntab[s]
        b = ebase + i * CA
        asum_tot = srsum_tot = None
        for g in range(CA // L):
            ln = _lanes(g)
            eg = (b + g * L) + lax.iota(jnp.int32, L)
            loc = eg // M - r0
            cbx = plsc.load_gather(tabn, [ln, _col(0)])
            cby = plsc.load_gather(tabn, [ln, _col(1)])
            cbz = plsc.load_gather(tabn, [ln, _col(2)])
            rb = [plsc.load_gather(tabn, [ln, _col(3 + j)])
                  for j in range(9)]
            nbx = plsc.load_gather(tabn, [ln, _col(12)])
            nby = plsc.load_gather(tabn, [ln, _col(13)])
            nbz = plsc.load_gather(tabn, [ln, _col(14)])
            cax = plsc.load_gather(srt, [loc, _col(0)])
            cay = plsc.load_gather(srt, [loc, _col(1)])
            caz = plsc.load_gather(srt, [loc, _col(2)])
            ra = [plsc.load_gather(srt, [loc, _col(3 + j)])
                  for j in range(9)]
            s9 = [ra[j] - rb[j] for j in range(9)]
            srsum = s9[0] * s9[0]
            for j in range(1, 9):
                srsum = srsum + s9[j] * s9[j]
            dx = (cax - cbx) + s9[0] * nbx + s9[1] * nby + s9[2] * nbz
            dy = (cay - cby) + s9[3] * nbx + s9[4] * nby + s9[5] * nbz
            dz = (caz - cbz) + s9[6] * nbx + s9[7] * nby + s9[8] * nbz
            asum = dx * dx + dy * dy + dz * dz
            mask = jnp.where(eg < E, 1.0, 0.0).astype(jnp.float32)
            if asum_tot is None:
                asum_tot, srsum_tot = mask * asum, mask * srsum
            else:
                asum_tot = asum_tot + mask * asum
                srsum_tot = srsum_tot + mask * srsum
        acc_a[...] += asum_tot
        acc_s[...] += srsum_tot

    nec = ET // CA
    arap_issue(0, 0)

    def arap_pair(j, carry):
        i = 2 * j
        arap_issue(i + 1, 1)
        arap_wait(0)
        arap_compute(i, 0)
        inext = jnp.minimum(i + 2, nec - 1)
        arap_issue(inext, 0)
        arap_wait(1)
        arap_compute(i + 1, 1)
        return carry

    lax.fori_loop(0, nec // 2, arap_pair, 0)
    arap_wait(0)
    pltpu.sync_copy(acc_a, lp_hbm.at[0, pl.ds(wid * L, L)])
    pltpu.sync_copy(acc_s, lp_hbm.at[1, pl.ds(wid * L, L)])


_main = functools.partial(
    pl.kernel,
    out_type=(
        jax.ShapeDtypeStruct((3, NP), jnp.float32),
        jax.ShapeDtypeStruct((2, NW * L), jnp.float32),
    ),
    mesh=_mesh,
    compiler_params=_sc_params,
    scratch_types=(
        [pltpu.VMEM((VT * 3,), jnp.int32)]
        + [pltpu.VMEM((ET,), jnp.int32)]
        + [pltpu.VMEM((CH * 3, 16), jnp.float32)] * 2
        + [pltpu.VMEM((CA, 16), jnp.float32)] * 2
        + [pltpu.VMEM((SRROWS, 16), jnp.float32)]
        + [pltpu.VMEM((VT,), jnp.float32)] * 9
        + [pltpu.VMEM((L,), jnp.float32)] * 2
        + [pltpu.SemaphoreType.DMA] * 4
    ),
)(_main_body)


def kernel(vertices, opt_d_rotations, opt_d_translations, weights,
           nodes_idx, influence_nodes_idx, one_ring_neigh):
    f32, i32 = jnp.float32, jnp.int32

    vp = jnp.pad(vertices.astype(f32), ((0, 0), (0, 1)))
    nidx = jnp.pad(nodes_idx.astype(i32), (0, NNP - NN))
    nrows = _gather_node_rows(vp, nidx)

    rt = jnp.pad(opt_d_rotations[0].astype(f32).T, ((0, 0), (0, NNP - NN)))
    tt = jnp.pad(opt_d_translations[0].astype(f32).T, ((0, 0), (0, NNP - NN)))
    nt = nrows.T[:3]
    table = _build_table(rt, tt, nt).T

    inflp = jnp.pad(influence_nodes_idx.astype(i32), ((0, NP - N), (0, 0)))
    wp = jnp.pad(weights.astype(f32), ((0, NP - N), (0, 0)))
    vtp = jnp.pad(vertices.astype(f32), ((0, NP - N), (0, 0)))
    whv = jnp.concatenate([wp.T, vtp.T], axis=0)
    orn = jnp.pad(one_ring_neigh.astype(i32).reshape(-1), (0, EP - E))
    ow, lp = _main(table, inflp.reshape(-1), whv, orn)

    warped = ow[:, :N].T[None]
    arap_loss = jnp.sum(lp[0]) / NN
    sr_loss = jnp.sum(lp[1]) / (E * 9)
    return (warped, arap_loss, sr_loss)


# speedup vs baseline: 1.0105x; 1.0105x over previous
import functools

import jax
import jax.numpy as jnp
from jax import lax
from jax.experimental import pallas as pl
from jax.experimental.pallas import tpu as pltpu
from jax.experimental.pallas import tpu_sc as plsc

N = 100000
NN = 50000
K = 3
M = 18
E = NN * M

NC, NS, L = 2, 16, 16
NW = NC * NS

CH = 128
CA = 128
NP = 106496
VT = NP // NW
NNP = 53248
NT = NNP // NW
EP = 901120
ET = EP // NW
SRROWS = 1600

_mesh = plsc.VectorSubcoreMesh(
    core_axis_name="c", subcore_axis_name="s", num_cores=NC, num_subcores=NS)
_sc_params = pltpu.CompilerParams(
    use_tc_tiling_on_sc=False, needs_layout_passes=False)


def _wid():
    return lax.axis_index("s") * NC + lax.axis_index("c")


@functools.partial(
    pl.kernel,
    out_type=jax.ShapeDtypeStruct((NNP, 16), jnp.float32),
    mesh=_mesh,
    compiler_params=_sc_params,
    scratch_types=[
        pltpu.VMEM((NT,), jnp.int32),
        pltpu.VMEM((CH, 16), jnp.float32),
        pltpu.VMEM((CH, 16), jnp.float32),
        pltpu.SemaphoreType.DMA,
        pltpu.SemaphoreType.DMA,
    ],
)
def _gather_node_rows(vp_hbm, nidx_hbm, out_hbm, idx_v, rva, rvb, sema, semb):
    base = _wid() * NT
    rows = [rva, rvb]
    sems = [sema, semb]
    pltpu.sync_copy(nidx_hbm.at[pl.ds(base, NT)], idx_v)
    nch = NT // CH

    def issue(i, s):
        pltpu.async_copy(vp_hbm.at[idx_v.at[pl.ds(i * CH, CH)]], rows[s],
                         sems[s])

    def drain(s):
        pltpu.make_async_copy(vp_hbm.at[idx_v.at[pl.ds(0, CH)]], rows[s],
                              sems[s]).wait()

    issue(0, 0)

    def body(j, carry):
        i = 2 * j
        issue(i + 1, 1)
        drain(0)
        pltpu.sync_copy(rows[0], out_hbm.at[pl.ds(base + i * CH, CH)])
        inext = jnp.minimum(i + 2, nch - 1)
        issue(inext, 0)
        drain(1)
        pltpu.sync_copy(rows[1], out_hbm.at[pl.ds(base + (i + 1) * CH, CH)])
        return carry

    lax.fori_loop(0, nch // 2, body, 0)
    drain(0)
    pltpu.sync_copy(rows[0], out_hbm.at[pl.ds(base + (nch - 1) * CH, CH)])


def _table_body(r_ref, t_ref, n_ref, o_ref):
    rx, ry, rz = r_ref[0:1, :], r_ref[1:2, :], r_ref[2:3, :]
    tx, ty, tz = t_ref[0:1, :], t_ref[1:2, :], t_ref[2:3, :]
    nx, ny, nz = n_ref[0:1, :], n_ref[1:2, :], n_ref[2:3, :]
    th = jnp.sqrt(rx * rx + ry * ry + rz * rz) + 1e-8
    inv = 1.0 / th
    x, y, z = rx * inv, ry * inv, rz * inv
    s = jnp.sin(th)
    cc = 1.0 - jnp.cos(th)
    r00 = 1.0 - cc * (y * y + z * z)
    r01 = -s * z + cc * x * y
    r02 = s * y + cc * x * z
    r10 = s * z + cc * x * y
    r11 = 1.0 - cc * (x * x + z * z)
    r12 = -s * x + cc * y * z
    r20 = -s * y + cc * x * z
    r21 = s * x + cc * y * z
    r22 = 1.0 - cc * (x * x + y * y)
    cx = nx + tx - (r00 * nx + r01 * ny + r02 * nz)
    cy = ny + ty - (r10 * nx + r11 * ny + r12 * nz)
    cz = nz + tz - (r20 * nx + r21 * ny + r22 * nz)
    cols = [cx, cy, cz, r00, r01, r02, r10, r11, r12, r20, r21, r22,
            nx, ny, nz, jnp.zeros_like(cx)]
    for i, col in enumerate(cols):
        o_ref[i:i + 1, :] = col


_build_table = pl.pallas_call(
    _table_body,
    out_shape=jax.ShapeDtypeStruct((16, NNP), jnp.float32),
)


def _lanes(g):
    return lax.iota(jnp.int32, L) + (g * L)


def _col(c):
    return jnp.full((L,), c, jnp.int32)


def _main_body(t_hbm, iw_hbm, whv_hbm, orn_hbm,
               ow_hbm, lp_hbm,
               ibw, nb,
               wtaba, wtabb,
               tabna, tabnb, srt,
               wb0, wb1, wb2, vbx, vby, vbz, obx, oby, obz,
               acc_a, acc_s, semwa, semwb, semna, semnb):
    wid = _wid()
    wtab = [wtaba, wtabb]
    wsem = [semwa, semwb]
    ntab = [tabna, tabnb]
    nsem = [semna, semnb]

    vbase = wid * VT
    pltpu.sync_copy(iw_hbm.at[pl.ds(vbase * 3, VT * 3)], ibw)
    pltpu.sync_copy(whv_hbm.at[pl.ds(0 * NP + vbase, VT)], wb0)
    pltpu.sync_copy(whv_hbm.at[pl.ds(1 * NP + vbase, VT)], wb1)
    pltpu.sync_copy(whv_hbm.at[pl.ds(2 * NP + vbase, VT)], wb2)
    pltpu.sync_copy(whv_hbm.at[pl.ds(3 * NP + vbase, VT)], vbx)
    pltpu.sync_copy(whv_hbm.at[pl.ds(4 * NP + vbase, VT)], vby)
    pltpu.sync_copy(whv_hbm.at[pl.ds(5 * NP + vbase, VT)], vbz)

    def warp_issue(i, s):
        pltpu.async_copy(t_hbm.at[ibw.at[pl.ds(i * CH * 3, CH * 3)]],
                         wtab[s], wsem[s])

    def warp_wait(s):
        pltpu.make_async_copy(t_hbm.at[ibw.at[pl.ds(0, CH * 3)]],
                              wtab[s], wsem[s]).wait()

    def warp_compute(i, s):
        tab = wtab[s]
        wbs = [wb0, wb1, wb2]
        for g in range(CH // L):
            ln = _lanes(g)
            off = i * CH + g * L
            a = [None] * 9
            bx = by = bz = None
            for k in range(3):
                lnk = ln * 3 + k
                w = wbs[k][pl.ds(off, L)]
                cx = plsc.load_gather(tab, [lnk, _col(0)])
                cy = plsc.load_gather(tab, [lnk, _col(1)])
                cz = plsc.load_gather(tab, [lnk, _col(2)])
                if k == 0:
                    bx, by, bz = w * cx, w * cy, w * cz
                else:
                    bx, by, bz = bx + w * cx, by + w * cy, bz + w * cz
                for j in range(9):
                    r = plsc.load_gather(tab, [lnk, _col(3 + j)])
                    a[j] = w * r if k == 0 else a[j] + w * r
            vx = vbx[pl.ds(off, L)]
            vy = vby[pl.ds(off, L)]
            vz = vbz[pl.ds(off, L)]
            obx[pl.ds(off, L)] = a[0] * vx + a[1] * vy + a[2] * vz + bx
            oby[pl.ds(off, L)] = a[3] * vx + a[4] * vy + a[5] * vz + by
            obz[pl.ds(off, L)] = a[6] * vx + a[7] * vy + a[8] * vz + bz

    nwc = VT // CH
    warp_issue(0, 0)

    def warp_pair(j, carry):
        i = 2 * j
        warp_issue(i + 1, 1)
        warp_wait(0)
        warp_compute(i, 0)
        inext = jnp.minimum(i + 2, nwc - 1)
        warp_issue(inext, 0)
        warp_wait(1)
        warp_compute(i + 1, 1)
        return carry

    lax.fori_loop(0, nwc // 2, warp_pair, 0)
    warp_wait(0)
    pltpu.sync_copy(obx, ow_hbm.at[pl.ds(0 * NP + vbase, VT)])
    pltpu.sync_copy(oby, ow_hbm.at[pl.ds(1 * NP + vbase, VT)])
    pltpu.sync_copy(obz, ow_hbm.at[pl.ds(2 * NP + vbase, VT)])

    acc_a[...] = jnp.zeros((L,), jnp.float32)
    acc_s[...] = jnp.zeros((L,), jnp.float32)
    ebase = wid * ET
    r0 = ebase // M
    pltpu.sync_copy(t_hbm.at[pl.ds(r0, SRROWS)], srt)
    pltpu.sync_copy(orn_hbm.at[pl.ds(ebase, ET)], nb)

    def arap_issue(i, s):
        pltpu.async_copy(t_hbm.at[nb.at[pl.ds(i * CA, CA)]], ntab[s],
                         nsem[s])

    def arap_wait(s):
        pltpu.make_async_copy(t_hbm.at[nb.at[pl.ds(0, CA)]], ntab[s],
                              nsem[s]).wait()

    def arap_compute(i, s):
        tabn = ntab[s]
        b = ebase + i * CA
        asum_tot = srsum_tot = None
        for g in range(CA // L):
            ln = _lanes(g)
            eg = (b + g * L) + lax.iota(jnp.int32, L)
            loc = eg // M - r0
            cbx = plsc.load_gather(tabn, [ln, _col(0)])
            cby = plsc.load_gather(tabn, [ln, _col(1)])
            cbz = plsc.load_gather(tabn, [ln, _col(2)])
            rb = [plsc.load_gather(tabn, [ln, _col(3 + j)])
                  for j in range(9)]
            nbx = plsc.load_gather(tabn, [ln, _col(12)])
            nby = plsc.load_gather(tabn, [ln, _col(13)])
            nbz = plsc.load_gather(tabn, [ln, _col(14)])
            cax = plsc.load_gather(srt, [loc, _col(0)])
            cay = plsc.load_gather(srt, [loc, _col(1)])
            caz = plsc.load_gather(srt, [loc, _col(2)])
            ra = [plsc.load_gather(srt, [loc, _col(3 + j)])
                  for j in range(9)]
            s9 = [ra[j] - rb[j] for j in range(9)]
            srsum = s9[0] * s9[0]
            for j in range(1, 9):
                srsum = srsum + s9[j] * s9[j]
            dx = (cax - cbx) + s9[0] * nbx + s9[1] * nby + s9[2] * nbz
            dy = (cay - cby) + s9[3] * nbx + s9[4] * nby + s9[5] * nbz
            dz = (caz - cbz) + s9[6] * nbx + s9[7] * nby + s9[8] * nbz
            asum = dx * dx + dy * dy + dz * dz
            mask = jnp.where(eg < E, 1.0, 0.0).astype(jnp.float32)
            if asum_tot is None:
                asum_tot, srsum_tot = mask * asum, mask * srsum
            else:
                asum_tot = asum_tot + mask * asum
                srsum_tot = srsum_tot + mask * srsum
        acc_a[...] += asum_tot
        acc_s[...] += srsum_tot

    nec = ET // CA
    arap_issue(0, 0)

    def arap_pair(j, carry):
        i = 2 * j
        arap_issue(i + 1, 1)
        arap_wait(0)
        arap_compute(i, 0)
        inext = jnp.minimum(i + 2, nec - 1)
        arap_issue(inext, 0)
        arap_wait(1)
        arap_compute(i + 1, 1)
        return carry

    lax.fori_loop(0, nec // 2, arap_pair, 0)
    arap_wait(0)
    pltpu.sync_copy(acc_a, lp_hbm.at[pl.ds(wid * L, L)])
    pltpu.sync_copy(acc_s, lp_hbm.at[pl.ds(NW * L + wid * L, L)])


_main = functools.partial(
    pl.kernel,
    out_type=(
        jax.ShapeDtypeStruct((3 * NP,), jnp.float32),
        jax.ShapeDtypeStruct((2 * NW * L,), jnp.float32),
    ),
    mesh=_mesh,
    compiler_params=_sc_params,
    scratch_types=(
        [pltpu.VMEM((VT * 3,), jnp.int32)]
        + [pltpu.VMEM((ET,), jnp.int32)]
        + [pltpu.VMEM((CH * 3, 16), jnp.float32)] * 2
        + [pltpu.VMEM((CA, 16), jnp.float32)] * 2
        + [pltpu.VMEM((SRROWS, 16), jnp.float32)]
        + [pltpu.VMEM((VT,), jnp.float32)] * 9
        + [pltpu.VMEM((L,), jnp.float32)] * 2
        + [pltpu.SemaphoreType.DMA] * 4
    ),
)(_main_body)


def kernel(vertices, opt_d_rotations, opt_d_translations, weights,
           nodes_idx, influence_nodes_idx, one_ring_neigh):
    f32, i32 = jnp.float32, jnp.int32

    vp = jnp.pad(vertices.astype(f32), ((0, 0), (0, 13)))
    nidx = jnp.pad(nodes_idx.astype(i32), (0, NNP - NN))
    nrows = _gather_node_rows(vp, nidx)

    rt = jnp.pad(opt_d_rotations[0].astype(f32).T, ((0, 0), (0, NNP - NN)))
    tt = jnp.pad(opt_d_translations[0].astype(f32).T, ((0, 0), (0, NNP - NN)))
    nt = nrows.T[:3]
    table = _build_table(rt, tt, nt).T

    inflp = jnp.pad(influence_nodes_idx.astype(i32), ((0, NP - N), (0, 0)))
    wp = jnp.pad(weights.astype(f32), ((0, NP - N), (0, 0)))
    vtp = jnp.pad(vertices.astype(f32), ((0, NP - N), (0, 0)))
    whv = jnp.concatenate([wp.T, vtp.T], axis=0).reshape(-1)
    orn = jnp.pad(one_ring_neigh.astype(i32).reshape(-1), (0, EP - E))
    ow, lp = _main(table, inflp.reshape(-1), whv, orn)

    warped = ow.reshape(3, NP)[:, :N].T[None]
    arap_loss = jnp.sum(lp[:NW * L]) / NN
    sr_loss = jnp.sum(lp[NW * L:]) / (E * 9)
    return (warped, arap_loss, sr_loss)


# speedup vs baseline: 1.1737x; 1.1615x over previous
import functools

import jax
import jax.numpy as jnp
from jax import lax
from jax.experimental import pallas as pl
from jax.experimental.pallas import tpu as pltpu
from jax.experimental.pallas import tpu_sc as plsc

N = 100000
NN = 50000
K = 3
M = 18
E = NN * M

NC, NS, L = 2, 16, 16
NW = NC * NS

CH = 128
NP = 106496
VT = NP // NW
NNP = 53248
NT = NNP // NW
EP = 901120
ET = EP // NW
SRROWS = 1600

_mesh = plsc.VectorSubcoreMesh(
    core_axis_name="c", subcore_axis_name="s", num_cores=NC, num_subcores=NS)
_sc_params = pltpu.CompilerParams(
    use_tc_tiling_on_sc=False, needs_layout_passes=False)


def _wid():
    return lax.axis_index("s") * NC + lax.axis_index("c")


@functools.partial(
    pl.kernel,
    out_type=jax.ShapeDtypeStruct((NNP, 16), jnp.float32),
    mesh=_mesh,
    compiler_params=_sc_params,
    scratch_types=[
        pltpu.VMEM((NT,), jnp.int32),
        pltpu.VMEM((CH, 16), jnp.float32),
        pltpu.VMEM((CH, 16), jnp.float32),
        pltpu.SemaphoreType.DMA,
        pltpu.SemaphoreType.DMA,
    ],
)
def _gather_node_rows(vp_hbm, nidx_hbm, out_hbm, idx_v, rva, rvb, sema, semb):
    base = _wid() * NT
    rows = [rva, rvb]
    sems = [sema, semb]
    pltpu.sync_copy(nidx_hbm.at[pl.ds(base, NT)], idx_v)
    nch = NT // CH

    def issue(i, s):
        pltpu.async_copy(vp_hbm.at[idx_v.at[pl.ds(i * CH, CH)]], rows[s],
                         sems[s])

    def drain(s):
        pltpu.make_async_copy(vp_hbm.at[idx_v.at[pl.ds(0, CH)]], rows[s],
                              sems[s]).wait()

    issue(0, 0)

    def body(j, carry):
        i = 2 * j
        issue(i + 1, 1)
        drain(0)
        pltpu.sync_copy(rows[0], out_hbm.at[pl.ds(base + i * CH, CH)])
        inext = jnp.minimum(i + 2, nch - 1)
        issue(inext, 0)
        drain(1)
        pltpu.sync_copy(rows[1], out_hbm.at[pl.ds(base + (i + 1) * CH, CH)])
        return carry

    lax.fori_loop(0, nch // 2, body, 0)
    drain(0)
    pltpu.sync_copy(rows[0], out_hbm.at[pl.ds(base + (nch - 1) * CH, CH)])


def _table_body(r_ref, t_ref, n_ref, o_ref):
    rx, ry, rz = r_ref[0:1, :], r_ref[1:2, :], r_ref[2:3, :]
    tx, ty, tz = t_ref[0:1, :], t_ref[1:2, :], t_ref[2:3, :]
    nx, ny, nz = n_ref[0:1, :], n_ref[1:2, :], n_ref[2:3, :]
    th = jnp.sqrt(rx * rx + ry * ry + rz * rz) + 1e-8
    inv = 1.0 / th
    x, y, z = rx * inv, ry * inv, rz * inv
    s = jnp.sin(th)
    cc = 1.0 - jnp.cos(th)
    r00 = 1.0 - cc * (y * y + z * z)
    r01 = -s * z + cc * x * y
    r02 = s * y + cc * x * z
    r10 = s * z + cc * x * y
    r11 = 1.0 - cc * (x * x + z * z)
    r12 = -s * x + cc * y * z
    r20 = -s * y + cc * x * z
    r21 = s * x + cc * y * z
    r22 = 1.0 - cc * (x * x + y * y)
    cx = nx + tx - (r00 * nx + r01 * ny + r02 * nz)
    cy = ny + ty - (r10 * nx + r11 * ny + r12 * nz)
    cz = nz + tz - (r20 * nx + r21 * ny + r22 * nz)
    cols = [cx, cy, cz, r00, r01, r02, r10, r11, r12, r20, r21, r22,
            nx, ny, nz, jnp.zeros_like(cx)]
    for i, col in enumerate(cols):
        o_ref[i:i + 1, :] = col


_build_table = pl.pallas_call(
    _table_body,
    out_shape=jax.ShapeDtypeStruct((16, NNP), jnp.float32),
)


def _lanes(g):
    return lax.iota(jnp.int32, L) + (g * L)


def _col(c):
    return jnp.full((L,), c, jnp.int32)


def _main_body(t_hbm, i0_hbm, i1_hbm, i2_hbm, w0_hbm, w1_hbm, w2_hbm,
               vx_hbm, vy_hbm, vz_hbm, orn_hbm,
               ox_hbm, oy_hbm, oz_hbm, ap_hbm, sp_hbm,
               ib0, ib1, ib2, nb,
               tab0a, tab1a, tab2a, tab0b, tab1b, tab2b,
               tabna, tabnb, srt,
               wb0, wb1, wb2, vbx, vby, vbz, obx, oby, obz,
               acc_a, acc_s, semwa, semwb, semna, semnb):
    wid = _wid()
    wtab = [[tab0a, tab1a, tab2a], [tab0b, tab1b, tab2b]]
    wsem = [semwa, semwb]
    ntab = [tabna, tabnb]
    nsem = [semna, semnb]
    ibs = [ib0, ib1, ib2]

    vbase = wid * VT
    pltpu.sync_copy(i0_hbm.at[pl.ds(vbase, VT)], ib0)
    pltpu.sync_copy(i1_hbm.at[pl.ds(vbase, VT)], ib1)
    pltpu.sync_copy(i2_hbm.at[pl.ds(vbase, VT)], ib2)
    pltpu.sync_copy(w0_hbm.at[pl.ds(vbase, VT)], wb0)
    pltpu.sync_copy(w1_hbm.at[pl.ds(vbase, VT)], wb1)
    pltpu.sync_copy(w2_hbm.at[pl.ds(vbase, VT)], wb2)
    pltpu.sync_copy(vx_hbm.at[pl.ds(vbase, VT)], vbx)
    pltpu.sync_copy(vy_hbm.at[pl.ds(vbase, VT)], vby)
    pltpu.sync_copy(vz_hbm.at[pl.ds(vbase, VT)], vbz)

    def warp_issue(i, s):
        for k in range(3):
            pltpu.async_copy(t_hbm.at[ibs[k].at[pl.ds(i * CH, CH)]],
                             wtab[s][k], wsem[s])

    def warp_wait(s):
        for k in range(3):
            pltpu.make_async_copy(t_hbm.at[ibs[k].at[pl.ds(0, CH)]],
                                  wtab[s][k], wsem[s]).wait()

    def warp_compute(i, s):
        tabs = wtab[s]
        wbs = [wb0, wb1, wb2]
        for g in range(CH // L):
            ln = _lanes(g)
            off = i * CH + g * L
            a = [None] * 9
            bx = by = bz = None
            for k in range(3):
                w = wbs[k][pl.ds(off, L)]
                cx = plsc.load_gather(tabs[k], [ln, _col(0)])
                cy = plsc.load_gather(tabs[k], [ln, _col(1)])
                cz = plsc.load_gather(tabs[k], [ln, _col(2)])
                if k == 0:
                    bx, by, bz = w * cx, w * cy, w * cz
                else:
                    bx, by, bz = bx + w * cx, by + w * cy, bz + w * cz
                for j in range(9):
                    r = plsc.load_gather(tabs[k], [ln, _col(3 + j)])
                    a[j] = w * r if k == 0 else a[j] + w * r
            vx = vbx[pl.ds(off, L)]
            vy = vby[pl.ds(off, L)]
            vz = vbz[pl.ds(off, L)]
            obx[pl.ds(off, L)] = a[0] * vx + a[1] * vy + a[2] * vz + bx
            oby[pl.ds(off, L)] = a[3] * vx + a[4] * vy + a[5] * vz + by
            obz[pl.ds(off, L)] = a[6] * vx + a[7] * vy + a[8] * vz + bz

    nwc = VT // CH
    warp_issue(0, 0)

    def warp_pair(j, carry):
        i = 2 * j
        warp_issue(i + 1, 1)
        warp_wait(0)
        warp_compute(i, 0)
        inext = jnp.minimum(i + 2, nwc - 1)
        warp_issue(inext, 0)
        warp_wait(1)
        warp_compute(i + 1, 1)
        return carry

    lax.fori_loop(0, nwc // 2, warp_pair, 0)
    warp_wait(0)
    pltpu.sync_copy(obx, ox_hbm.at[pl.ds(vbase, VT)])
    pltpu.sync_copy(oby, oy_hbm.at[pl.ds(vbase, VT)])
    pltpu.sync_copy(obz, oz_hbm.at[pl.ds(vbase, VT)])

    acc_a[...] = jnp.zeros((L,), jnp.float32)
    acc_s[...] = jnp.zeros((L,), jnp.float32)
    ebase = wid * ET
    r0 = ebase // M
    pltpu.sync_copy(t_hbm.at[pl.ds(r0, SRROWS)], srt)
    pltpu.sync_copy(orn_hbm.at[pl.ds(ebase, ET)], nb)

    def arap_issue(i, s):
        pltpu.async_copy(t_hbm.at[nb.at[pl.ds(i * CH, CH)]], ntab[s],
                         nsem[s])

    def arap_wait(s):
        pltpu.make_async_copy(t_hbm.at[nb.at[pl.ds(0, CH)]], ntab[s],
                              nsem[s]).wait()

    def arap_compute(i, s):
        tabn = ntab[s]
        b = ebase + i * CH
        asum_tot = srsum_tot = None
        for g in range(CH // L):
            ln = _lanes(g)
            eg = (b + g * L) + lax.iota(jnp.int32, L)
            loc = eg // M - r0
            cbx = plsc.load_gather(tabn, [ln, _col(0)])
            cby = plsc.load_gather(tabn, [ln, _col(1)])
            cbz = plsc.load_gather(tabn, [ln, _col(2)])
            rb = [plsc.load_gather(tabn, [ln, _col(3 + j)])
                  for j in range(9)]
            nbx = plsc.load_gather(tabn, [ln, _col(12)])
            nby = plsc.load_gather(tabn, [ln, _col(13)])
            nbz = plsc.load_gather(tabn, [ln, _col(14)])
            cax = plsc.load_gather(srt, [loc, _col(0)])
            cay = plsc.load_gather(srt, [loc, _col(1)])
            caz = plsc.load_gather(srt, [loc, _col(2)])
            ra = [plsc.load_gather(srt, [loc, _col(3 + j)])
                  for j in range(9)]
            s9 = [ra[j] - rb[j] for j in range(9)]
            srsum = s9[0] * s9[0]
            for j in range(1, 9):
                srsum = srsum + s9[j] * s9[j]
            dx = (cax - cbx) + s9[0] * nbx + s9[1] * nby + s9[2] * nbz
            dy = (cay - cby) + s9[3] * nbx + s9[4] * nby + s9[5] * nbz
            dz = (caz - cbz) + s9[6] * nbx + s9[7] * nby + s9[8] * nbz
            asum = dx * dx + dy * dy + dz * dz
            mask = jnp.where(eg < E, 1.0, 0.0).astype(jnp.float32)
            if asum_tot is None:
                asum_tot, srsum_tot = mask * asum, mask * srsum
            else:
                asum_tot = asum_tot + mask * asum
                srsum_tot = srsum_tot + mask * srsum
        acc_a[...] += asum_tot
        acc_s[...] += srsum_tot

    nec = ET // CH
    arap_issue(0, 0)

    def arap_pair(j, carry):
        i = 2 * j
        arap_issue(i + 1, 1)
        arap_wait(0)
        arap_compute(i, 0)
        inext = jnp.minimum(i + 2, nec - 1)
        arap_issue(inext, 0)
        arap_wait(1)
        arap_compute(i + 1, 1)
        return carry

    lax.fori_loop(0, nec // 2, arap_pair, 0)
    arap_wait(0)
    pltpu.sync_copy(acc_a, ap_hbm.at[pl.ds(wid * L, L)])
    pltpu.sync_copy(acc_s, sp_hbm.at[pl.ds(wid * L, L)])


_main = functools.partial(
    pl.kernel,
    out_type=(
        jax.ShapeDtypeStruct((NP,), jnp.float32),
        jax.ShapeDtypeStruct((NP,), jnp.float32),
        jax.ShapeDtypeStruct((NP,), jnp.float32),
        jax.ShapeDtypeStruct((NW * L,), jnp.float32),
        jax.ShapeDtypeStruct((NW * L,), jnp.float32),
    ),
    mesh=_mesh,
    compiler_params=_sc_params,
    scratch_types=(
        [pltpu.VMEM((VT,), jnp.int32)] * 3
        + [pltpu.VMEM((ET,), jnp.int32)]
        + [pltpu.VMEM((CH, 16), jnp.float32)] * 6
        + [pltpu.VMEM((CH, 16), jnp.float32)] * 2
        + [pltpu.VMEM((SRROWS, 16), jnp.float32)]
        + [pltpu.VMEM((VT,), jnp.float32)] * 9
        + [pltpu.VMEM((L,), jnp.float32)] * 2
        + [pltpu.SemaphoreType.DMA] * 4
    ),
)(_main_body)


def kernel(vertices, opt_d_rotations, opt_d_translations, weights,
           nodes_idx, influence_nodes_idx, one_ring_neigh):
    f32, i32 = jnp.float32, jnp.int32

    vp = jnp.pad(vertices.astype(f32), ((0, 0), (0, 13)))
    nidx = jnp.pad(nodes_idx.astype(i32), (0, NNP - NN))
    nrows = _gather_node_rows(vp, nidx)

    rt = jnp.pad(opt_d_rotations[0].astype(f32).T, ((0, 0), (0, NNP - NN)))
    tt = jnp.pad(opt_d_translations[0].astype(f32).T, ((0, 0), (0, NNP - NN)))
    nt = nrows.T[:3]
    table = _build_table(rt, tt, nt).T

    inflp = jnp.pad(influence_nodes_idx.astype(i32), ((0, NP - N), (0, 0)))
    wp = jnp.pad(weights.astype(f32), ((0, NP - N), (0, 0)))
    vtp = jnp.pad(vertices.astype(f32), ((0, NP - N), (0, 0)))
    orn = jnp.pad(one_ring_neigh.astype(i32).reshape(-1), (0, EP - E))
    ox, oy, oz, ap, sp = _main(
        table, inflp[:, 0], inflp[:, 1], inflp[:, 2],
        wp[:, 0], wp[:, 1], wp[:, 2],
        vtp[:, 0], vtp[:, 1], vtp[:, 2], orn)

    warped = jnp.stack([ox, oy, oz], axis=1)[:N][None]
    arap_loss = jnp.sum(ap) / NN
    sr_loss = jnp.sum(sp) / (E * 9)
    return (warped, arap_loss, sr_loss)
